# Initial kernel scaffold; baseline (speedup 1.0000x reference)
import jax
import jax.numpy as jnp
from jax.experimental import pallas as pl


def kernel(x_user, x_item, edge_index_u2i, edge_index_i2u, Wl_u2i, bl_u2i, Wr_u2i, Wl_i2u, bl_i2u, Wr_i2u, gamma_user, beta_user, gamma_item, beta_item):
    raise NotImplementedError("write your pallas kernel here")


# speedup vs baseline: 3.0885x; 3.0885x over previous
import jax
import jax.numpy as jnp
from jax import lax
from jax.experimental import pallas as pl
from jax.experimental.pallas import tpu as pltpu
from jax.experimental.pallas import tpu_sc as plsc

N = 10000
D = 256
E = 160000
EPS = 1e-5

NS = 16
L = 16
DH = 128
SH_ROWS = 10368
DUMP = 10240
OUT_ROWS = 10240
EPT = E // NS
CHUNK = 64
SEG = 2048
ZR_TILE = SH_ROWS // NS
WR_TILE = OUT_ROWS // NS
STAGES = ((2048, 2048), (2048, 2048), (2048, 2048), (2048, 2048),
          (1808, 1856))


def _sc_agg_kernel(xu2, xi2, su2i, du2i, si2u, di2u,
                   aggi_lo, aggi_hi, cnti0, cnti1,
                   aggu_lo, aggu_hi, cntu0, cntu1,
                   acc_sh, zrow, ones,
                   src_raw, dst_raw, rows, gidx, sidx, sem):
    cid = lax.axis_index("c")
    sid = lax.axis_index("s")
    z16 = jnp.zeros((L,), jnp.float32)
    zi16 = jnp.zeros((L,), jnp.int32)
    one16 = jnp.ones((L,), jnp.float32)
    dump16 = jnp.full((L,), DUMP, jnp.int32)

    def zrow_body(i, _):
        zrow[lax.rem(i, 8), pl.ds(lax.div(i, 8) * L, L)] = z16
        return 0
    lax.fori_loop(0, 8 * (DH // L), zrow_body, 0)

    def ones_body(i, _):
        ones[lax.rem(i, CHUNK), pl.ds(lax.div(i, CHUNK) * L, L)] = one16
        return 0
    lax.fori_loop(0, CHUNK * (DH // L), ones_body, 0)

    def zero_acc():
        def z_body(t, _):
            pltpu.sync_copy(zrow, acc_sh.at[pl.ds(sid * ZR_TILE + t * 8, 8)])
            return 0
        lax.fori_loop(0, ZR_TILE // 8, z_body, 0)

    for (x2, s_hbm, d_hbm, out_lo, out_hi, out_c0, out_c1) in (
            (xu2, su2i, du2i, aggi_lo, aggi_hi, cnti0, cnti1),
            (xi2, si2u, di2u, aggu_lo, aggu_hi, cntu0, cntu1)):
        zero_acc()
        plsc.subcore_barrier()

        def ch_body(j, _):
            base = j * CHUNK
            for t in range(CHUNK // L):
                s = src_raw[pl.ds(base + t * L, L)]
                d = dst_raw[pl.ds(base + t * L, L)]
                gidx[pl.ds(t * L, L)] = 2 * s + cid
                sidx[pl.ds(t * L, L)] = d
            pltpu.async_copy(x2.at[gidx], rows, sem).wait()
            pltpu.sync_copy(rows, acc_sh.at[sidx], add=True)
            return 0

        for st, (n_real, n_pad) in enumerate(STAGES):
            e0 = sid * EPT + st * SEG
            pltpu.sync_copy(s_hbm.at[pl.ds(e0, n_real)],
                            src_raw.at[pl.ds(0, n_real)])
            pltpu.sync_copy(d_hbm.at[pl.ds(e0, n_real)],
                            dst_raw.at[pl.ds(0, n_real)])
            for t in range((n_pad - n_real) // L):
                src_raw[pl.ds(n_real + t * L, L)] = zi16
                dst_raw[pl.ds(n_real + t * L, L)] = dump16
            lax.fori_loop(0, n_pad // CHUNK, ch_body, 0)

        plsc.subcore_barrier()

        r0 = sid * WR_TILE

        @pl.when(cid == 0)
        def _():
            pltpu.sync_copy(acc_sh.at[pl.ds(r0, WR_TILE)],
                            out_lo.at[pl.ds(r0, WR_TILE)])

        @pl.when(cid == 1)
        def _():
            pltpu.sync_copy(acc_sh.at[pl.ds(r0, WR_TILE)],
                            out_hi.at[pl.ds(r0, WR_TILE)])

        plsc.subcore_barrier()

        zero_acc()
        plsc.subcore_barrier()

        def cnt_body(j, _):
            base = j * CHUNK
            for t in range(CHUNK // L):
                d = dst_raw[pl.ds(base + t * L, L)]
                sidx[pl.ds(t * L, L)] = d

            @pl.when(lax.rem(j, 2) == cid)
            def _():
                pltpu.sync_copy(ones, acc_sh.at[sidx], add=True)
            return 0

        for st, (n_real, n_pad) in enumerate(STAGES):
            e0 = sid * EPT + st * SEG
            pltpu.sync_copy(d_hbm.at[pl.ds(e0, n_real)],
                            dst_raw.at[pl.ds(0, n_real)])
            for t in range((n_pad - n_real) // L):
                dst_raw[pl.ds(n_real + t * L, L)] = dump16
            lax.fori_loop(0, n_pad // CHUNK, cnt_body, 0)

        plsc.subcore_barrier()

        @pl.when(cid == 0)
        def _():
            pltpu.sync_copy(acc_sh.at[pl.ds(r0, WR_TILE)],
                            out_c0.at[pl.ds(r0, WR_TILE)])

        @pl.when(cid == 1)
        def _():
            pltpu.sync_copy(acc_sh.at[pl.ds(r0, WR_TILE)],
                            out_c1.at[pl.ds(r0, WR_TILE)])

        plsc.subcore_barrier()


def _sc_agg(xu2, xi2, su2i, du2i, si2u, di2u):
    mesh = plsc.VectorSubcoreMesh(core_axis_name="c", subcore_axis_name="s")
    f32 = jnp.float32
    out_type = (
        jax.ShapeDtypeStruct((OUT_ROWS, DH), f32),
        jax.ShapeDtypeStruct((OUT_ROWS, DH), f32),
        jax.ShapeDtypeStruct((OUT_ROWS, DH), f32),
        jax.ShapeDtypeStruct((OUT_ROWS, DH), f32),
        jax.ShapeDtypeStruct((OUT_ROWS, DH), f32),
        jax.ShapeDtypeStruct((OUT_ROWS, DH), f32),
        jax.ShapeDtypeStruct((OUT_ROWS, DH), f32),
        jax.ShapeDtypeStruct((OUT_ROWS, DH), f32),
    )
    scratch = [
        pltpu.VMEM_SHARED((SH_ROWS, DH), f32),
        pltpu.VMEM((8, DH), f32),
        pltpu.VMEM((CHUNK, DH), f32),
        pltpu.VMEM((SEG,), jnp.int32),
        pltpu.VMEM((SEG,), jnp.int32),
        pltpu.VMEM((CHUNK, DH), f32),
        pltpu.VMEM((CHUNK,), jnp.int32),
        pltpu.VMEM((CHUNK,), jnp.int32),
        pltpu.SemaphoreType.DMA,
    ]
    fn = pl.kernel(_sc_agg_kernel, out_type=out_type, mesh=mesh,
                   scratch_types=scratch)
    return fn(xu2, xi2, su2i, du2i, si2u, di2u)


BM = 400
NB = N // BM


def _phase_a_kernel(alo_ref, ahi_ref, c0_ref, c1_ref, x_ref, wll_ref, wlh_ref,
                    wr_ref, bl_ref, r_ref, stats_ref):
    b = pl.program_id(0)
    cnt = c0_ref[...][:, 0:1] + c1_ref[...][:, 0:1]
    denom = jnp.maximum(cnt, 1.0)
    ml = alo_ref[...] / denom
    mh = ahi_ref[...] / denom
    h = (jnp.dot(ml, wll_ref[...], preferred_element_type=jnp.float32)
         + jnp.dot(mh, wlh_ref[...], preferred_element_type=jnp.float32)
         + jnp.dot(x_ref[...], wr_ref[...], preferred_element_type=jnp.float32)
         + bl_ref[...])
    r = jnp.maximum(h, 0.0)
    r_ref[...] = r

    @pl.when(b == 0)
    def _():
        stats_ref[...] = jnp.zeros_like(stats_ref)

    stats_ref[0:1, :] += jnp.sum(r, axis=0, keepdims=True)
    stats_ref[1:2, :] += jnp.sum(r * r, axis=0, keepdims=True)


def _phase_b_kernel(r_ref, stats_ref, gamma_ref, beta_ref, out_ref):
    m = stats_ref[0:1, :] / N
    var = stats_ref[1:2, :] / N - m * m
    inv = lax.rsqrt(var + EPS)
    scale = gamma_ref[...] * inv
    shift = beta_ref[...] - m * scale
    out_ref[...] = r_ref[...] * scale + shift


def _tc_post(agg_lo, agg_hi, cnt0, cnt1, x, wl, wr, bl, gamma, beta):
    f32 = jnp.float32
    wl_lo = wl[:DH]
    wl_hi = wl[DH:]
    r, stats = pl.pallas_call(
        _phase_a_kernel,
        grid=(NB,),
        in_specs=[
            pl.BlockSpec((BM, DH), lambda b: (b, 0)),
            pl.BlockSpec((BM, DH), lambda b: (b, 0)),
            pl.BlockSpec((BM, DH), lambda b: (b, 0)),
            pl.BlockSpec((BM, DH), lambda b: (b, 0)),
            pl.BlockSpec((BM, D), lambda b: (b, 0)),
            pl.BlockSpec((DH, D), lambda b: (0, 0)),
            pl.BlockSpec((DH, D), lambda b: (0, 0)),
            pl.BlockSpec((D, D), lambda b: (0, 0)),
            pl.BlockSpec((1, D), lambda b: (0, 0)),
        ],
        out_specs=[
            pl.BlockSpec((BM, D), lambda b: (b, 0)),
            pl.BlockSpec((8, D), lambda b: (0, 0)),
        ],
        out_shape=[
            jax.ShapeDtypeStruct((N, D), f32),
            jax.ShapeDtypeStruct((8, D), f32),
        ],
    )(agg_lo, agg_hi, cnt0, cnt1, x, wl_lo, wl_hi, wr, bl)
    out = pl.pallas_call(
        _phase_b_kernel,
        grid=(NB,),
        in_specs=[
            pl.BlockSpec((BM, D), lambda b: (b, 0)),
            pl.BlockSpec((8, D), lambda b: (0, 0)),
            pl.BlockSpec((1, D), lambda b: (0, 0)),
            pl.BlockSpec((1, D), lambda b: (0, 0)),
        ],
        out_specs=pl.BlockSpec((BM, D), lambda b: (b, 0)),
        out_shape=jax.ShapeDtypeStruct((N, D), f32),
    )(r, stats, gamma, beta)
    return out


def kernel(x_user, x_item, edge_index_u2i, edge_index_i2u,
           Wl_u2i, bl_u2i, Wr_u2i, Wl_i2u, bl_i2u, Wr_i2u,
           gamma_user, beta_user, gamma_item, beta_item):
    su2i, du2i = edge_index_u2i[0], edge_index_u2i[1]
    si2u, di2u = edge_index_i2u[0], edge_index_i2u[1]
    xu2 = x_user.reshape(2 * N, DH)
    xi2 = x_item.reshape(2 * N, DH)

    (aggi_lo, aggi_hi, cnti0, cnti1,
     aggu_lo, aggu_hi, cntu0, cntu1) = _sc_agg(
        xu2, xi2, su2i, du2i, si2u, di2u)

    out_item = _tc_post(aggi_lo, aggi_hi, cnti0, cnti1, x_item,
                        Wl_u2i, Wr_u2i, bl_u2i.reshape(1, D),
                        gamma_item.reshape(1, D), beta_item.reshape(1, D))
    out_user = _tc_post(aggu_lo, aggu_hi, cntu0, cntu1, x_user,
                        Wl_i2u, Wr_i2u, bl_i2u.reshape(1, D),
                        gamma_user.reshape(1, D), beta_user.reshape(1, D))
    return (out_user, out_item)


# speedup vs baseline: 3.3514x; 1.0851x over previous
import jax
import jax.numpy as jnp
from jax import lax
from jax.experimental import pallas as pl
from jax.experimental.pallas import tpu as pltpu
from jax.experimental.pallas import tpu_sc as plsc

N = 10000
D = 256
E = 160000
EPS = 1e-5

NS = 16
L = 16
DH = 128
SH_ROWS = 10368
DUMP = 10240
OUT_ROWS = 10240
EPT = E // NS
CHUNK = 96
SEG = 1920
ZR_TILE = SH_ROWS // NS
WR_TILE = OUT_ROWS // NS
STAGES = ((1920, 1920), (1920, 1920), (1920, 1920), (1920, 1920),
          (1920, 1920), (400, 576))


def _sc_agg_kernel(xu2, xi2, su2i, du2i, si2u, di2u,
                   aggi_lo, aggi_hi, cnti0, cnti1,
                   aggu_lo, aggu_hi, cntu0, cntu1,
                   acc_sh, zrow, ones,
                   src_raw, dst_raw, rows_a, rows_b,
                   gidx_a, gidx_b, sidx_a, sidx_b, sem_a, sem_b):
    cid = lax.axis_index("c")
    sid = lax.axis_index("s")
    z16 = jnp.zeros((L,), jnp.float32)
    zi16 = jnp.zeros((L,), jnp.int32)
    one16 = jnp.ones((L,), jnp.float32)
    dump16 = jnp.full((L,), DUMP, jnp.int32)

    def zrow_body(i, _):
        zrow[lax.rem(i, 8), pl.ds(lax.div(i, 8) * L, L)] = z16
        return 0
    lax.fori_loop(0, 8 * (DH // L), zrow_body, 0)

    def ones_body(i, _):
        ones[lax.rem(i, CHUNK), pl.ds(lax.div(i, CHUNK) * L, L)] = one16
        return 0
    lax.fori_loop(0, CHUNK * (DH // L), ones_body, 0)

    def zero_acc():
        def z_body(t, _):
            pltpu.sync_copy(zrow, acc_sh.at[pl.ds(sid * ZR_TILE + t * 8, 8)])
            return 0
        lax.fori_loop(0, ZR_TILE // 8, z_body, 0)

    for (x2, s_hbm, d_hbm, out_lo, out_hi, out_c0, out_c1) in (
            (xu2, su2i, du2i, aggi_lo, aggi_hi, cnti0, cnti1),
            (xi2, si2u, di2u, aggu_lo, aggu_hi, cntu0, cntu1)):
        zero_acc()
        plsc.subcore_barrier()

        def build_idx(base, gidx, sidx):
            for t in range(CHUNK // L):
                s = src_raw[pl.ds(base + t * L, L)]
                d = dst_raw[pl.ds(base + t * L, L)]
                gidx[pl.ds(t * L, L)] = 2 * s + cid
                sidx[pl.ds(t * L, L)] = d

        def gather_a():
            return pltpu.make_async_copy(x2.at[gidx_a], rows_a, sem_a)

        def gather_b():
            return pltpu.make_async_copy(x2.at[gidx_b], rows_b, sem_b)

        for st, (n_real, n_pad) in enumerate(STAGES):
            e0 = sid * EPT + st * SEG
            pltpu.sync_copy(s_hbm.at[pl.ds(e0, n_real)],
                            src_raw.at[pl.ds(0, n_real)])
            pltpu.sync_copy(d_hbm.at[pl.ds(e0, n_real)],
                            dst_raw.at[pl.ds(0, n_real)])
            for t in range((n_pad - n_real) // L):
                src_raw[pl.ds(n_real + t * L, L)] = zi16
                dst_raw[pl.ds(n_real + t * L, L)] = dump16

            npairs = n_pad // (2 * CHUNK)
            build_idx(0, gidx_a, sidx_a)
            gather_a().start()

            def pair_body(p, _):
                build_idx((2 * p + 1) * CHUNK, gidx_b, sidx_b)
                gather_b().start()
                gather_a().wait()
                pltpu.sync_copy(rows_a, acc_sh.at[sidx_a], add=True)

                @pl.when(p < npairs - 1)
                def _():
                    build_idx((2 * p + 2) * CHUNK, gidx_a, sidx_a)
                    gather_a().start()

                gather_b().wait()
                pltpu.sync_copy(rows_b, acc_sh.at[sidx_b], add=True)
                return 0
            lax.fori_loop(0, npairs, pair_body, 0)

        plsc.subcore_barrier()

        r0 = sid * WR_TILE

        @pl.when(cid == 0)
        def _():
            pltpu.sync_copy(acc_sh.at[pl.ds(r0, WR_TILE)],
                            out_lo.at[pl.ds(r0, WR_TILE)])

        @pl.when(cid == 1)
        def _():
            pltpu.sync_copy(acc_sh.at[pl.ds(r0, WR_TILE)],
                            out_hi.at[pl.ds(r0, WR_TILE)])

        plsc.subcore_barrier()

        zero_acc()
        plsc.subcore_barrier()

        def cnt_body(j, _):
            base = (2 * j + cid) * CHUNK
            for t in range(CHUNK // L):
                d = dst_raw[pl.ds(base + t * L, L)]
                sidx_a[pl.ds(t * L, L)] = d
            pltpu.sync_copy(ones, acc_sh.at[sidx_a], add=True)
            return 0

        for st, (n_real, n_pad) in enumerate(STAGES):
            e0 = sid * EPT + st * SEG
            pltpu.sync_copy(d_hbm.at[pl.ds(e0, n_real)],
                            dst_raw.at[pl.ds(0, n_real)])
            for t in range((n_pad - n_real) // L):
                dst_raw[pl.ds(n_real + t * L, L)] = dump16
            lax.fori_loop(0, n_pad // (2 * CHUNK), cnt_body, 0)

        plsc.subcore_barrier()

        @pl.when(cid == 0)
        def _():
            pltpu.sync_copy(acc_sh.at[pl.ds(r0, WR_TILE)],
                            out_c0.at[pl.ds(r0, WR_TILE)])

        @pl.when(cid == 1)
        def _():
            pltpu.sync_copy(acc_sh.at[pl.ds(r0, WR_TILE)],
                            out_c1.at[pl.ds(r0, WR_TILE)])

        plsc.subcore_barrier()


def _sc_agg(xu2, xi2, su2i, du2i, si2u, di2u):
    mesh = plsc.VectorSubcoreMesh(core_axis_name="c", subcore_axis_name="s")
    f32 = jnp.float32
    out_type = (
        jax.ShapeDtypeStruct((OUT_ROWS, DH), f32),
        jax.ShapeDtypeStruct((OUT_ROWS, DH), f32),
        jax.ShapeDtypeStruct((OUT_ROWS, DH), f32),
        jax.ShapeDtypeStruct((OUT_ROWS, DH), f32),
        jax.ShapeDtypeStruct((OUT_ROWS, DH), f32),
        jax.ShapeDtypeStruct((OUT_ROWS, DH), f32),
        jax.ShapeDtypeStruct((OUT_ROWS, DH), f32),
        jax.ShapeDtypeStruct((OUT_ROWS, DH), f32),
    )
    scratch = [
        pltpu.VMEM_SHARED((SH_ROWS, DH), f32),
        pltpu.VMEM((8, DH), f32),
        pltpu.VMEM((CHUNK, DH), f32),
        pltpu.VMEM((SEG,), jnp.int32),
        pltpu.VMEM((SEG,), jnp.int32),
        pltpu.VMEM((CHUNK, DH), f32),
        pltpu.VMEM((CHUNK, DH), f32),
        pltpu.VMEM((CHUNK,), jnp.int32),
        pltpu.VMEM((CHUNK,), jnp.int32),
        pltpu.VMEM((CHUNK,), jnp.int32),
        pltpu.VMEM((CHUNK,), jnp.int32),
        pltpu.SemaphoreType.DMA,
        pltpu.SemaphoreType.DMA,
    ]
    fn = pl.kernel(_sc_agg_kernel, out_type=out_type, mesh=mesh,
                   scratch_types=scratch)
    return fn(xu2, xi2, su2i, du2i, si2u, di2u)


BM = 400
NB = N // BM


def _phase_a_kernel(alo_ref, ahi_ref, c0_ref, c1_ref, x_ref, wll_ref, wlh_ref,
                    wr_ref, bl_ref, r_ref, stats_ref):
    b = pl.program_id(0)
    cnt = c0_ref[...][:, 0:1] + c1_ref[...][:, 0:1]
    denom = jnp.maximum(cnt, 1.0)
    ml = alo_ref[...] / denom
    mh = ahi_ref[...] / denom
    h = (jnp.dot(ml, wll_ref[...], preferred_element_type=jnp.float32)
         + jnp.dot(mh, wlh_ref[...], preferred_element_type=jnp.float32)
         + jnp.dot(x_ref[...], wr_ref[...], preferred_element_type=jnp.float32)
         + bl_ref[...])
    r = jnp.maximum(h, 0.0)
    r_ref[...] = r

    @pl.when(b == 0)
    def _():
        stats_ref[...] = jnp.zeros_like(stats_ref)

    stats_ref[0:1, :] += jnp.sum(r, axis=0, keepdims=True)
    stats_ref[1:2, :] += jnp.sum(r * r, axis=0, keepdims=True)


def _phase_b_kernel(r_ref, stats_ref, gamma_ref, beta_ref, out_ref):
    m = stats_ref[0:1, :] / N
    var = stats_ref[1:2, :] / N - m * m
    inv = lax.rsqrt(var + EPS)
    scale = gamma_ref[...] * inv
    shift = beta_ref[...] - m * scale
    out_ref[...] = r_ref[...] * scale + shift


def _tc_post(agg_lo, agg_hi, cnt0, cnt1, x, wl, wr, bl, gamma, beta):
    f32 = jnp.float32
    wl_lo = wl[:DH]
    wl_hi = wl[DH:]
    r, stats = pl.pallas_call(
        _phase_a_kernel,
        grid=(NB,),
        in_specs=[
            pl.BlockSpec((BM, DH), lambda b: (b, 0)),
            pl.BlockSpec((BM, DH), lambda b: (b, 0)),
            pl.BlockSpec((BM, DH), lambda b: (b, 0)),
            pl.BlockSpec((BM, DH), lambda b: (b, 0)),
            pl.BlockSpec((BM, D), lambda b: (b, 0)),
            pl.BlockSpec((DH, D), lambda b: (0, 0)),
            pl.BlockSpec((DH, D), lambda b: (0, 0)),
            pl.BlockSpec((D, D), lambda b: (0, 0)),
            pl.BlockSpec((1, D), lambda b: (0, 0)),
        ],
        out_specs=[
            pl.BlockSpec((BM, D), lambda b: (b, 0)),
            pl.BlockSpec((8, D), lambda b: (0, 0)),
        ],
        out_shape=[
            jax.ShapeDtypeStruct((N, D), f32),
            jax.ShapeDtypeStruct((8, D), f32),
        ],
    )(agg_lo, agg_hi, cnt0, cnt1, x, wl_lo, wl_hi, wr, bl)
    out = pl.pallas_call(
        _phase_b_kernel,
        grid=(NB,),
        in_specs=[
            pl.BlockSpec((BM, D), lambda b: (b, 0)),
            pl.BlockSpec((8, D), lambda b: (0, 0)),
            pl.BlockSpec((1, D), lambda b: (0, 0)),
            pl.BlockSpec((1, D), lambda b: (0, 0)),
        ],
        out_specs=pl.BlockSpec((BM, D), lambda b: (b, 0)),
        out_shape=jax.ShapeDtypeStruct((N, D), f32),
    )(r, stats, gamma, beta)
    return out


def kernel(x_user, x_item, edge_index_u2i, edge_index_i2u,
           Wl_u2i, bl_u2i, Wr_u2i, Wl_i2u, bl_i2u, Wr_i2u,
           gamma_user, beta_user, gamma_item, beta_item):
    su2i, du2i = edge_index_u2i[0], edge_index_u2i[1]
    si2u, di2u = edge_index_i2u[0], edge_index_i2u[1]
    xu2 = x_user.reshape(2 * N, DH)
    xi2 = x_item.reshape(2 * N, DH)

    (aggi_lo, aggi_hi, cnti0, cnti1,
     aggu_lo, aggu_hi, cntu0, cntu1) = _sc_agg(
        xu2, xi2, su2i, du2i, si2u, di2u)

    out_item = _tc_post(aggi_lo, aggi_hi, cnti0, cnti1, x_item,
                        Wl_u2i, Wr_u2i, bl_u2i.reshape(1, D),
                        gamma_item.reshape(1, D), beta_item.reshape(1, D))
    out_user = _tc_post(aggu_lo, aggu_hi, cntu0, cntu1, x_user,
                        Wl_i2u, Wr_i2u, bl_i2u.reshape(1, D),
                        gamma_user.reshape(1, D), beta_user.reshape(1, D))
    return (out_user, out_item)


# speedup vs baseline: 3.3540x; 1.0008x over previous
import jax
import jax.numpy as jnp
from jax import lax
from jax.experimental import pallas as pl
from jax.experimental.pallas import tpu as pltpu
from jax.experimental.pallas import tpu_sc as plsc

N = 10000
D = 256
E = 160000
EPS = 1e-5

NS = 16
L = 16
DH = 128
SH_ROWS = 10368
DUMP = 10240
OUT_ROWS = 10240
EPT = E // NS
CHUNK = 96
SEG = 1920
ZR_TILE = SH_ROWS // NS
WR_TILE = OUT_ROWS // NS
STAGES = ((1920, 1920), (1920, 1920), (1920, 1920), (1920, 1920),
          (1920, 1920), (400, 576))


def _sc_agg_kernel(xu2, xi2, su2i, du2i, si2u, di2u,
                   aggi_lo, aggi_hi, cnti0, cnti1,
                   aggu_lo, aggu_hi, cntu0, cntu1,
                   acc_sh, zrow, ones,
                   src_raw, dst_raw, rows_a, rows_b,
                   gidx_a, gidx_b, sidx_a, sidx_b,
                   sem_a, sem_b, ssem_a, ssem_b):
    cid = lax.axis_index("c")
    sid = lax.axis_index("s")
    z16 = jnp.zeros((L,), jnp.float32)
    zi16 = jnp.zeros((L,), jnp.int32)
    one16 = jnp.ones((L,), jnp.float32)
    dump16 = jnp.full((L,), DUMP, jnp.int32)

    def zrow_body(i, _):
        zrow[lax.rem(i, 8), pl.ds(lax.div(i, 8) * L, L)] = z16
        return 0
    lax.fori_loop(0, 8 * (DH // L), zrow_body, 0)

    def ones_body(i, _):
        ones[lax.rem(i, CHUNK), pl.ds(lax.div(i, CHUNK) * L, L)] = one16
        return 0
    lax.fori_loop(0, CHUNK * (DH // L), ones_body, 0)

    def zero_acc():
        def z_body(t, _):
            pltpu.sync_copy(zrow, acc_sh.at[pl.ds(sid * ZR_TILE + t * 8, 8)])
            return 0
        lax.fori_loop(0, ZR_TILE // 8, z_body, 0)

    for (x2, s_hbm, d_hbm, out_lo, out_hi, out_c0, out_c1) in (
            (xu2, su2i, du2i, aggi_lo, aggi_hi, cnti0, cnti1),
            (xi2, si2u, di2u, aggu_lo, aggu_hi, cntu0, cntu1)):
        zero_acc()
        plsc.subcore_barrier()

        def build_idx(base, gidx, sidx):
            for t in range(CHUNK // L):
                s = src_raw[pl.ds(base + t * L, L)]
                d = dst_raw[pl.ds(base + t * L, L)]
                gidx[pl.ds(t * L, L)] = 2 * s + cid
                sidx[pl.ds(t * L, L)] = d

        def gather_a():
            return pltpu.make_async_copy(x2.at[gidx_a], rows_a, sem_a)

        def gather_b():
            return pltpu.make_async_copy(x2.at[gidx_b], rows_b, sem_b)

        def scatter_a_start():
            pltpu.async_copy(rows_a, acc_sh.at[sidx_a], ssem_a, add=True)

        def scatter_a_wait():
            pltpu.make_async_copy(rows_a, acc_sh.at[sidx_a], ssem_a).wait()

        def scatter_b_start():
            pltpu.async_copy(rows_b, acc_sh.at[sidx_b], ssem_b, add=True)

        def scatter_b_wait():
            pltpu.make_async_copy(rows_b, acc_sh.at[sidx_b], ssem_b).wait()

        for st, (n_real, n_pad) in enumerate(STAGES):
            e0 = sid * EPT + st * SEG
            pltpu.sync_copy(s_hbm.at[pl.ds(e0, n_real)],
                            src_raw.at[pl.ds(0, n_real)])
            pltpu.sync_copy(d_hbm.at[pl.ds(e0, n_real)],
                            dst_raw.at[pl.ds(0, n_real)])
            for t in range((n_pad - n_real) // L):
                src_raw[pl.ds(n_real + t * L, L)] = zi16
                dst_raw[pl.ds(n_real + t * L, L)] = dump16

            npairs = n_pad // (2 * CHUNK)
            build_idx(0, gidx_a, sidx_a)
            gather_a().start()

            def pair_body(p, _):
                @pl.when(p > 0)
                def _():
                    scatter_b_wait()
                build_idx((2 * p + 1) * CHUNK, gidx_b, sidx_b)
                gather_b().start()
                gather_a().wait()
                scatter_a_start()

                @pl.when(p < npairs - 1)
                def _():
                    scatter_a_wait()
                    build_idx((2 * p + 2) * CHUNK, gidx_a, sidx_a)
                    gather_a().start()

                gather_b().wait()
                scatter_b_start()
                return 0
            lax.fori_loop(0, npairs, pair_body, 0)
            scatter_a_wait()
            scatter_b_wait()

        plsc.subcore_barrier()

        r0 = sid * WR_TILE

        @pl.when(cid == 0)
        def _():
            pltpu.sync_copy(acc_sh.at[pl.ds(r0, WR_TILE)],
                            out_lo.at[pl.ds(r0, WR_TILE)])

        @pl.when(cid == 1)
        def _():
            pltpu.sync_copy(acc_sh.at[pl.ds(r0, WR_TILE)],
                            out_hi.at[pl.ds(r0, WR_TILE)])

        plsc.subcore_barrier()

        zero_acc()
        plsc.subcore_barrier()

        def cscat_a_start():
            pltpu.async_copy(ones, acc_sh.at[sidx_a], ssem_a, add=True)

        def cscat_a_wait():
            pltpu.make_async_copy(ones, acc_sh.at[sidx_a], ssem_a).wait()

        def cscat_b_start():
            pltpu.async_copy(ones, acc_sh.at[sidx_b], ssem_b, add=True)

        def cscat_b_wait():
            pltpu.make_async_copy(ones, acc_sh.at[sidx_b], ssem_b).wait()

        def build_didx(base, sidx):
            for t in range(CHUNK // L):
                sidx[pl.ds(t * L, L)] = dst_raw[pl.ds(base + t * L, L)]

        def cnt_body(m, _):
            even = lax.rem(m, 2) == 0

            @pl.when(jnp.logical_and(m > 1, even))
            def _():
                cscat_a_wait()

            @pl.when(jnp.logical_and(m > 1, jnp.logical_not(even)))
            def _():
                cscat_b_wait()

            @pl.when(even)
            def _():
                build_didx((2 * m + cid) * CHUNK, sidx_a)
                cscat_a_start()

            @pl.when(jnp.logical_not(even))
            def _():
                build_didx((2 * m + cid) * CHUNK, sidx_b)
                cscat_b_start()
            return 0

        for st, (n_real, n_pad) in enumerate(STAGES):
            e0 = sid * EPT + st * SEG
            pltpu.sync_copy(d_hbm.at[pl.ds(e0, n_real)],
                            dst_raw.at[pl.ds(0, n_real)])
            for t in range((n_pad - n_real) // L):
                dst_raw[pl.ds(n_real + t * L, L)] = dump16
            nmine = n_pad // (2 * CHUNK)
            lax.fori_loop(0, nmine, cnt_body, 0)
            cscat_a_wait()
            if nmine >= 2:
                cscat_b_wait()

        plsc.subcore_barrier()

        @pl.when(cid == 0)
        def _():
            pltpu.sync_copy(acc_sh.at[pl.ds(r0, WR_TILE)],
                            out_c0.at[pl.ds(r0, WR_TILE)])

        @pl.when(cid == 1)
        def _():
            pltpu.sync_copy(acc_sh.at[pl.ds(r0, WR_TILE)],
                            out_c1.at[pl.ds(r0, WR_TILE)])

        plsc.subcore_barrier()


def _sc_agg(xu2, xi2, su2i, du2i, si2u, di2u):
    mesh = plsc.VectorSubcoreMesh(core_axis_name="c", subcore_axis_name="s")
    f32 = jnp.float32
    out_type = (
        jax.ShapeDtypeStruct((OUT_ROWS, DH), f32),
        jax.ShapeDtypeStruct((OUT_ROWS, DH), f32),
        jax.ShapeDtypeStruct((OUT_ROWS, DH), f32),
        jax.ShapeDtypeStruct((OUT_ROWS, DH), f32),
        jax.ShapeDtypeStruct((OUT_ROWS, DH), f32),
        jax.ShapeDtypeStruct((OUT_ROWS, DH), f32),
        jax.ShapeDtypeStruct((OUT_ROWS, DH), f32),
        jax.ShapeDtypeStruct((OUT_ROWS, DH), f32),
    )
    scratch = [
        pltpu.VMEM_SHARED((SH_ROWS, DH), f32),
        pltpu.VMEM((8, DH), f32),
        pltpu.VMEM((CHUNK, DH), f32),
        pltpu.VMEM((SEG,), jnp.int32),
        pltpu.VMEM((SEG,), jnp.int32),
        pltpu.VMEM((CHUNK, DH), f32),
        pltpu.VMEM((CHUNK, DH), f32),
        pltpu.VMEM((CHUNK,), jnp.int32),
        pltpu.VMEM((CHUNK,), jnp.int32),
        pltpu.VMEM((CHUNK,), jnp.int32),
        pltpu.VMEM((CHUNK,), jnp.int32),
        pltpu.SemaphoreType.DMA,
        pltpu.SemaphoreType.DMA,
        pltpu.SemaphoreType.DMA,
        pltpu.SemaphoreType.DMA,
    ]
    fn = pl.kernel(_sc_agg_kernel, out_type=out_type, mesh=mesh,
                   scratch_types=scratch)
    return fn(xu2, xi2, su2i, du2i, si2u, di2u)


BM = 400
NB = N // BM


def _phase_a_kernel(alo_ref, ahi_ref, c0_ref, c1_ref, x_ref, wll_ref, wlh_ref,
                    wr_ref, bl_ref, r_ref, stats_ref):
    b = pl.program_id(0)
    cnt = c0_ref[...][:, 0:1] + c1_ref[...][:, 0:1]
    denom = jnp.maximum(cnt, 1.0)
    ml = alo_ref[...] / denom
    mh = ahi_ref[...] / denom
    h = (jnp.dot(ml, wll_ref[...], preferred_element_type=jnp.float32)
         + jnp.dot(mh, wlh_ref[...], preferred_element_type=jnp.float32)
         + jnp.dot(x_ref[...], wr_ref[...], preferred_element_type=jnp.float32)
         + bl_ref[...])
    r = jnp.maximum(h, 0.0)
    r_ref[...] = r

    @pl.when(b == 0)
    def _():
        stats_ref[...] = jnp.zeros_like(stats_ref)

    stats_ref[0:1, :] += jnp.sum(r, axis=0, keepdims=True)
    stats_ref[1:2, :] += jnp.sum(r * r, axis=0, keepdims=True)


def _phase_b_kernel(r_ref, stats_ref, gamma_ref, beta_ref, out_ref):
    m = stats_ref[0:1, :] / N
    var = stats_ref[1:2, :] / N - m * m
    inv = lax.rsqrt(var + EPS)
    scale = gamma_ref[...] * inv
    shift = beta_ref[...] - m * scale
    out_ref[...] = r_ref[...] * scale + shift


def _tc_post(agg_lo, agg_hi, cnt0, cnt1, x, wl, wr, bl, gamma, beta):
    f32 = jnp.float32
    wl_lo = wl[:DH]
    wl_hi = wl[DH:]
    r, stats = pl.pallas_call(
        _phase_a_kernel,
        grid=(NB,),
        in_specs=[
            pl.BlockSpec((BM, DH), lambda b: (b, 0)),
            pl.BlockSpec((BM, DH), lambda b: (b, 0)),
            pl.BlockSpec((BM, DH), lambda b: (b, 0)),
            pl.BlockSpec((BM, DH), lambda b: (b, 0)),
            pl.BlockSpec((BM, D), lambda b: (b, 0)),
            pl.BlockSpec((DH, D), lambda b: (0, 0)),
            pl.BlockSpec((DH, D), lambda b: (0, 0)),
            pl.BlockSpec((D, D), lambda b: (0, 0)),
            pl.BlockSpec((1, D), lambda b: (0, 0)),
        ],
        out_specs=[
            pl.BlockSpec((BM, D), lambda b: (b, 0)),
            pl.BlockSpec((8, D), lambda b: (0, 0)),
        ],
        out_shape=[
            jax.ShapeDtypeStruct((N, D), f32),
            jax.ShapeDtypeStruct((8, D), f32),
        ],
    )(agg_lo, agg_hi, cnt0, cnt1, x, wl_lo, wl_hi, wr, bl)
    out = pl.pallas_call(
        _phase_b_kernel,
        grid=(NB,),
        in_specs=[
            pl.BlockSpec((BM, D), lambda b: (b, 0)),
            pl.BlockSpec((8, D), lambda b: (0, 0)),
            pl.BlockSpec((1, D), lambda b: (0, 0)),
            pl.BlockSpec((1, D), lambda b: (0, 0)),
        ],
        out_specs=pl.BlockSpec((BM, D), lambda b: (b, 0)),
        out_shape=jax.ShapeDtypeStruct((N, D), f32),
    )(r, stats, gamma, beta)
    return out


def kernel(x_user, x_item, edge_index_u2i, edge_index_i2u,
           Wl_u2i, bl_u2i, Wr_u2i, Wl_i2u, bl_i2u, Wr_i2u,
           gamma_user, beta_user, gamma_item, beta_item):
    su2i, du2i = edge_index_u2i[0], edge_index_u2i[1]
    si2u, di2u = edge_index_i2u[0], edge_index_i2u[1]
    xu2 = x_user.reshape(2 * N, DH)
    xi2 = x_item.reshape(2 * N, DH)

    (aggi_lo, aggi_hi, cnti0, cnti1,
     aggu_lo, aggu_hi, cntu0, cntu1) = _sc_agg(
        xu2, xi2, su2i, du2i, si2u, di2u)

    out_item = _tc_post(aggi_lo, aggi_hi, cnti0, cnti1, x_item,
                        Wl_u2i, Wr_u2i, bl_u2i.reshape(1, D),
                        gamma_item.reshape(1, D), beta_item.reshape(1, D))
    out_user = _tc_post(aggu_lo, aggu_hi, cntu0, cntu1, x_user,
                        Wl_i2u, Wr_i2u, bl_i2u.reshape(1, D),
                        gamma_user.reshape(1, D), beta_user.reshape(1, D))
    return (out_user, out_item)


# speedup vs baseline: 3.5432x; 1.0564x over previous
import jax
import jax.numpy as jnp
from jax import lax
from jax.experimental import pallas as pl
from jax.experimental.pallas import tpu as pltpu
from jax.experimental.pallas import tpu_sc as plsc

N = 10000
D = 256
E = 160000
EPS = 1e-5

NS = 16
L = 16
DH = 128
SH_ROWS = 10368
DUMP = 10240
OUT_ROWS = 10240
EPT = E // NS
CHUNK = 96
SEG = 1920
ZR_TILE = SH_ROWS // NS
WR_TILE = OUT_ROWS // NS
STAGES = ((1920, 1920), (1920, 1920), (1920, 1920), (1920, 1920),
          (1920, 1920), (400, 576))


def _sc_agg_kernel(x2, s_hbm, d_hbm,
                   agg_lo_o, agg_hi_o, cnt0_o, cnt1_o,
                   acc_sh, zrow, ones,
                   src_raw, dst_raw, rows_a, rows_b,
                   gidx_a, gidx_b, sidx_a, sidx_b,
                   sem_a, sem_b, ssem_a, ssem_b):
    cid = lax.axis_index("c")
    sid = lax.axis_index("s")
    z16 = jnp.zeros((L,), jnp.float32)
    zi16 = jnp.zeros((L,), jnp.int32)
    one16 = jnp.ones((L,), jnp.float32)
    dump16 = jnp.full((L,), DUMP, jnp.int32)

    def zrow_body(i, _):
        zrow[lax.rem(i, 8), pl.ds(lax.div(i, 8) * L, L)] = z16
        return 0
    lax.fori_loop(0, 8 * (DH // L), zrow_body, 0)

    def ones_body(i, _):
        ones[lax.rem(i, CHUNK), pl.ds(lax.div(i, CHUNK) * L, L)] = one16
        return 0
    lax.fori_loop(0, CHUNK * (DH // L), ones_body, 0)

    def zero_acc():
        def z_body(t, _):
            pltpu.sync_copy(zrow, acc_sh.at[pl.ds(sid * ZR_TILE + t * 8, 8)])
            return 0
        lax.fori_loop(0, ZR_TILE // 8, z_body, 0)

    for (out_lo, out_hi, out_c0, out_c1) in (
            (agg_lo_o, agg_hi_o, cnt0_o, cnt1_o),):
        zero_acc()
        plsc.subcore_barrier()

        def build_idx(base, gidx, sidx):
            for t in range(CHUNK // L):
                s = src_raw[pl.ds(base + t * L, L)]
                d = dst_raw[pl.ds(base + t * L, L)]
                gidx[pl.ds(t * L, L)] = 2 * s + cid
                sidx[pl.ds(t * L, L)] = d

        def gather_a():
            return pltpu.make_async_copy(x2.at[gidx_a], rows_a, sem_a)

        def gather_b():
            return pltpu.make_async_copy(x2.at[gidx_b], rows_b, sem_b)

        def scatter_a_start():
            pltpu.async_copy(rows_a, acc_sh.at[sidx_a], ssem_a, add=True)

        def scatter_a_wait():
            pltpu.make_async_copy(rows_a, acc_sh.at[sidx_a], ssem_a).wait()

        def scatter_b_start():
            pltpu.async_copy(rows_b, acc_sh.at[sidx_b], ssem_b, add=True)

        def scatter_b_wait():
            pltpu.make_async_copy(rows_b, acc_sh.at[sidx_b], ssem_b).wait()

        for st, (n_real, n_pad) in enumerate(STAGES):
            e0 = sid * EPT + st * SEG
            pltpu.sync_copy(s_hbm.at[pl.ds(e0, n_real)],
                            src_raw.at[pl.ds(0, n_real)])
            pltpu.sync_copy(d_hbm.at[pl.ds(e0, n_real)],
                            dst_raw.at[pl.ds(0, n_real)])
            for t in range((n_pad - n_real) // L):
                src_raw[pl.ds(n_real + t * L, L)] = zi16
                dst_raw[pl.ds(n_real + t * L, L)] = dump16

            npairs = n_pad // (2 * CHUNK)
            build_idx(0, gidx_a, sidx_a)
            gather_a().start()

            def pair_body(p, _):
                @pl.when(p > 0)
                def _():
                    scatter_b_wait()
                build_idx((2 * p + 1) * CHUNK, gidx_b, sidx_b)
                gather_b().start()
                gather_a().wait()
                scatter_a_start()

                @pl.when(p < npairs - 1)
                def _():
                    scatter_a_wait()
                    build_idx((2 * p + 2) * CHUNK, gidx_a, sidx_a)
                    gather_a().start()

                gather_b().wait()
                scatter_b_start()
                return 0
            lax.fori_loop(0, npairs, pair_body, 0)
            scatter_a_wait()
            scatter_b_wait()

        plsc.subcore_barrier()

        r0 = sid * WR_TILE

        @pl.when(cid == 0)
        def _():
            pltpu.sync_copy(acc_sh.at[pl.ds(r0, WR_TILE)],
                            out_lo.at[pl.ds(r0, WR_TILE)])

        @pl.when(cid == 1)
        def _():
            pltpu.sync_copy(acc_sh.at[pl.ds(r0, WR_TILE)],
                            out_hi.at[pl.ds(r0, WR_TILE)])

        plsc.subcore_barrier()

        zero_acc()
        plsc.subcore_barrier()

        def cscat_a_start():
            pltpu.async_copy(ones, acc_sh.at[sidx_a], ssem_a, add=True)

        def cscat_a_wait():
            pltpu.make_async_copy(ones, acc_sh.at[sidx_a], ssem_a).wait()

        def cscat_b_start():
            pltpu.async_copy(ones, acc_sh.at[sidx_b], ssem_b, add=True)

        def cscat_b_wait():
            pltpu.make_async_copy(ones, acc_sh.at[sidx_b], ssem_b).wait()

        def build_didx(base, sidx):
            for t in range(CHUNK // L):
                sidx[pl.ds(t * L, L)] = dst_raw[pl.ds(base + t * L, L)]

        def cnt_body(m, _):
            even = lax.rem(m, 2) == 0

            @pl.when(jnp.logical_and(m > 1, even))
            def _():
                cscat_a_wait()

            @pl.when(jnp.logical_and(m > 1, jnp.logical_not(even)))
            def _():
                cscat_b_wait()

            @pl.when(even)
            def _():
                build_didx((2 * m + cid) * CHUNK, sidx_a)
                cscat_a_start()

            @pl.when(jnp.logical_not(even))
            def _():
                build_didx((2 * m + cid) * CHUNK, sidx_b)
                cscat_b_start()
            return 0

        for st, (n_real, n_pad) in enumerate(STAGES):
            e0 = sid * EPT + st * SEG
            pltpu.sync_copy(d_hbm.at[pl.ds(e0, n_real)],
                            dst_raw.at[pl.ds(0, n_real)])
            for t in range((n_pad - n_real) // L):
                dst_raw[pl.ds(n_real + t * L, L)] = dump16
            nmine = n_pad // (2 * CHUNK)
            lax.fori_loop(0, nmine, cnt_body, 0)
            cscat_a_wait()
            if nmine >= 2:
                cscat_b_wait()

        plsc.subcore_barrier()

        @pl.when(cid == 0)
        def _():
            pltpu.sync_copy(acc_sh.at[pl.ds(r0, WR_TILE)],
                            out_c0.at[pl.ds(r0, WR_TILE)])

        @pl.when(cid == 1)
        def _():
            pltpu.sync_copy(acc_sh.at[pl.ds(r0, WR_TILE)],
                            out_c1.at[pl.ds(r0, WR_TILE)])

        plsc.subcore_barrier()


def _sc_agg(x2, s_hbm, d_hbm):
    mesh = plsc.VectorSubcoreMesh(core_axis_name="c", subcore_axis_name="s")
    f32 = jnp.float32
    out_type = (
        jax.ShapeDtypeStruct((OUT_ROWS, DH), f32),
        jax.ShapeDtypeStruct((OUT_ROWS, DH), f32),
        jax.ShapeDtypeStruct((OUT_ROWS, DH), f32),
        jax.ShapeDtypeStruct((OUT_ROWS, DH), f32),
    )
    scratch = [
        pltpu.VMEM_SHARED((SH_ROWS, DH), f32),
        pltpu.VMEM((8, DH), f32),
        pltpu.VMEM((CHUNK, DH), f32),
        pltpu.VMEM((SEG,), jnp.int32),
        pltpu.VMEM((SEG,), jnp.int32),
        pltpu.VMEM((CHUNK, DH), f32),
        pltpu.VMEM((CHUNK, DH), f32),
        pltpu.VMEM((CHUNK,), jnp.int32),
        pltpu.VMEM((CHUNK,), jnp.int32),
        pltpu.VMEM((CHUNK,), jnp.int32),
        pltpu.VMEM((CHUNK,), jnp.int32),
        pltpu.SemaphoreType.DMA,
        pltpu.SemaphoreType.DMA,
        pltpu.SemaphoreType.DMA,
        pltpu.SemaphoreType.DMA,
    ]
    fn = pl.kernel(_sc_agg_kernel, out_type=out_type, mesh=mesh,
                   scratch_types=scratch)
    return fn(x2, s_hbm, d_hbm)


BM = 400
NB = N // BM


def _tc_fused_kernel(alo_ref, ahi_ref, c0_ref, c1_ref, x_ref, wll_ref,
                     wlh_ref, wr_ref, bl_ref, gamma_ref, beta_ref,
                     out_ref, r_ref, stats_ref):
    p = pl.program_id(0)
    b = pl.program_id(1)

    @pl.when(p == 0)
    def _():
        cnt = c0_ref[...][:, 0:1] + c1_ref[...][:, 0:1]
        denom = jnp.maximum(cnt, 1.0)
        ml = alo_ref[...] / denom
        mh = ahi_ref[...] / denom
        h = (jnp.dot(ml, wll_ref[...], preferred_element_type=jnp.float32)
             + jnp.dot(mh, wlh_ref[...], preferred_element_type=jnp.float32)
             + jnp.dot(x_ref[...], wr_ref[...],
                       preferred_element_type=jnp.float32)
             + bl_ref[...])
        r = jnp.maximum(h, 0.0)
        r_ref[pl.ds(b * BM, BM), :] = r

        @pl.when(b == 0)
        def _():
            stats_ref[...] = jnp.zeros_like(stats_ref)

        stats_ref[0:1, :] += jnp.sum(r, axis=0, keepdims=True)
        stats_ref[1:2, :] += jnp.sum(r * r, axis=0, keepdims=True)

    @pl.when(p == 1)
    def _():
        m = stats_ref[0:1, :] / N
        var = stats_ref[1:2, :] / N - m * m
        inv = lax.rsqrt(var + EPS)
        scale = gamma_ref[...] * inv
        shift = beta_ref[...] - m * scale
        out_ref[...] = r_ref[pl.ds(b * BM, BM), :] * scale + shift


def _tc_post(agg_lo, agg_hi, cnt0, cnt1, x, wl, wr, bl, gamma, beta):
    f32 = jnp.float32
    wl_lo = wl[:DH]
    wl_hi = wl[DH:]
    out = pl.pallas_call(
        _tc_fused_kernel,
        grid=(2, NB),
        in_specs=[
            pl.BlockSpec((BM, DH), lambda p, b: (b, 0)),
            pl.BlockSpec((BM, DH), lambda p, b: (b, 0)),
            pl.BlockSpec((BM, DH), lambda p, b: (b, 0)),
            pl.BlockSpec((BM, DH), lambda p, b: (b, 0)),
            pl.BlockSpec((BM, D), lambda p, b: (b, 0)),
            pl.BlockSpec((DH, D), lambda p, b: (0, 0)),
            pl.BlockSpec((DH, D), lambda p, b: (0, 0)),
            pl.BlockSpec((D, D), lambda p, b: (0, 0)),
            pl.BlockSpec((1, D), lambda p, b: (0, 0)),
            pl.BlockSpec((1, D), lambda p, b: (0, 0)),
            pl.BlockSpec((1, D), lambda p, b: (0, 0)),
        ],
        out_specs=pl.BlockSpec((BM, D), lambda p, b: (b, 0)),
        out_shape=jax.ShapeDtypeStruct((N, D), f32),
        scratch_shapes=[
            pltpu.VMEM((N, D), f32),
            pltpu.VMEM((8, D), f32),
        ],
    )(agg_lo, agg_hi, cnt0, cnt1, x, wl_lo, wl_hi, wr, bl, gamma, beta)
    return out


def kernel(x_user, x_item, edge_index_u2i, edge_index_i2u,
           Wl_u2i, bl_u2i, Wr_u2i, Wl_i2u, bl_i2u, Wr_i2u,
           gamma_user, beta_user, gamma_item, beta_item):
    su2i, du2i = edge_index_u2i[0], edge_index_u2i[1]
    si2u, di2u = edge_index_i2u[0], edge_index_i2u[1]
    xu2 = x_user.reshape(2 * N, DH)
    xi2 = x_item.reshape(2 * N, DH)

    aggi_lo, aggi_hi, cnti0, cnti1 = _sc_agg(xu2, su2i, du2i)
    aggu_lo, aggu_hi, cntu0, cntu1 = _sc_agg(xi2, si2u, di2u)

    out_item = _tc_post(aggi_lo, aggi_hi, cnti0, cnti1, x_item,
                        Wl_u2i, Wr_u2i, bl_u2i.reshape(1, D),
                        gamma_item.reshape(1, D), beta_item.reshape(1, D))
    out_user = _tc_post(aggu_lo, aggu_hi, cntu0, cntu1, x_user,
                        Wl_i2u, Wr_i2u, bl_i2u.reshape(1, D),
                        gamma_user.reshape(1, D), beta_user.reshape(1, D))
    return (out_user, out_item)


# speedup vs baseline: 3.7262x; 1.0516x over previous
import jax
import jax.numpy as jnp
from jax import lax
from jax.experimental import pallas as pl
from jax.experimental.pallas import tpu as pltpu
from jax.experimental.pallas import tpu_sc as plsc

N = 10000
D = 256
E = 160000
EPS = 1e-5

NS = 16
L = 16
DH = 128
SH_ROWS = 10368
DUMP = 10240
OUT_ROWS = 10240
EPT = E // NS
CHUNK = 96
SEG = 1920
ZR_TILE = SH_ROWS // NS
WR_TILE = OUT_ROWS // NS
STAGES = ((1920, 1920), (1920, 1920), (1920, 1920), (1920, 1920),
          (1920, 1920), (400, 576))


def _sc_agg_kernel(x2, s_hbm, d_hbm,
                   agg_lo_o, agg_hi_o, cnt0_o, cnt1_o,
                   acc_sh, zrow, ones,
                   src_raw, dst_raw, rows_a, rows_b,
                   gidx_a, gidx_b, sidx_a, sidx_b,
                   sem_a, sem_b, ssem_a, ssem_b):
    cid = lax.axis_index("c")
    sid = lax.axis_index("s")
    z16 = jnp.zeros((L,), jnp.float32)
    zi16 = jnp.zeros((L,), jnp.int32)
    one16 = jnp.ones((L,), jnp.float32)
    dump16 = jnp.full((L,), DUMP, jnp.int32)

    def zrow_body(i, _):
        zrow[lax.rem(i, 8), pl.ds(lax.div(i, 8) * L, L)] = z16
        return 0
    lax.fori_loop(0, 8 * (DH // L), zrow_body, 0)

    def ones_body(i, _):
        ones[lax.rem(i, CHUNK), pl.ds(lax.div(i, CHUNK) * L, L)] = one16
        return 0
    lax.fori_loop(0, CHUNK * (DH // L), ones_body, 0)

    def zero_acc():
        def z_body(t, _):
            pltpu.sync_copy(zrow, acc_sh.at[pl.ds(sid * ZR_TILE + t * 8, 8)])
            return 0
        lax.fori_loop(0, ZR_TILE // 8, z_body, 0)

    for (out_lo, out_hi, out_c0, out_c1) in (
            (agg_lo_o, agg_hi_o, cnt0_o, cnt1_o),):
        zero_acc()
        plsc.subcore_barrier()

        def build_idx(base, gidx, sidx):
            for t in range(CHUNK // L):
                s = src_raw[pl.ds(base + t * L, L)]
                d = dst_raw[pl.ds(base + t * L, L)]
                gidx[pl.ds(t * L, L)] = 2 * s + cid
                sidx[pl.ds(t * L, L)] = d

        def gather_a():
            return pltpu.make_async_copy(x2.at[gidx_a], rows_a, sem_a)

        def gather_b():
            return pltpu.make_async_copy(x2.at[gidx_b], rows_b, sem_b)

        def scatter_a_start():
            pltpu.async_copy(rows_a, acc_sh.at[sidx_a], ssem_a, add=True)

        def scatter_a_wait():
            pltpu.make_async_copy(rows_a, acc_sh.at[sidx_a], ssem_a).wait()

        def scatter_b_start():
            pltpu.async_copy(rows_b, acc_sh.at[sidx_b], ssem_b, add=True)

        def scatter_b_wait():
            pltpu.make_async_copy(rows_b, acc_sh.at[sidx_b], ssem_b).wait()

        for st, (n_real, n_pad) in enumerate(STAGES):
            e0 = sid * EPT + st * SEG
            pltpu.sync_copy(s_hbm.at[pl.ds(e0, n_real)],
                            src_raw.at[pl.ds(0, n_real)])
            pltpu.sync_copy(d_hbm.at[pl.ds(e0, n_real)],
                            dst_raw.at[pl.ds(0, n_real)])
            for t in range((n_pad - n_real) // L):
                src_raw[pl.ds(n_real + t * L, L)] = zi16
                dst_raw[pl.ds(n_real + t * L, L)] = dump16

            npairs = n_pad // (2 * CHUNK)
            build_idx(0, gidx_a, sidx_a)
            gather_a().start()

            def pair_body(p, _):
                @pl.when(p > 0)
                def _():
                    scatter_b_wait()
                build_idx((2 * p + 1) * CHUNK, gidx_b, sidx_b)
                gather_b().start()
                gather_a().wait()
                scatter_a_start()

                @pl.when(p < npairs - 1)
                def _():
                    scatter_a_wait()
                    build_idx((2 * p + 2) * CHUNK, gidx_a, sidx_a)
                    gather_a().start()

                gather_b().wait()
                scatter_b_start()
                return 0
            lax.fori_loop(0, npairs, pair_body, 0)
            scatter_a_wait()
            scatter_b_wait()

        plsc.subcore_barrier()

        r0 = sid * WR_TILE

        @pl.when(cid == 0)
        def _():
            pltpu.sync_copy(acc_sh.at[pl.ds(r0, WR_TILE)],
                            out_lo.at[pl.ds(r0, WR_TILE)])

        @pl.when(cid == 1)
        def _():
            pltpu.sync_copy(acc_sh.at[pl.ds(r0, WR_TILE)],
                            out_hi.at[pl.ds(r0, WR_TILE)])

        plsc.subcore_barrier()


        def cscat_a_start():
            pltpu.async_copy(ones, acc_sh.at[sidx_a], ssem_a, add=True)

        def cscat_a_wait():
            pltpu.make_async_copy(ones, acc_sh.at[sidx_a], ssem_a).wait()

        def cscat_b_start():
            pltpu.async_copy(ones, acc_sh.at[sidx_b], ssem_b, add=True)

        def cscat_b_wait():
            pltpu.make_async_copy(ones, acc_sh.at[sidx_b], ssem_b).wait()

        def build_didx(base, sidx):
            for t in range(CHUNK // L):
                sidx[pl.ds(t * L, L)] = dst_raw[pl.ds(base + t * L, L)]

        def cnt_body(m, _):
            even = lax.rem(m, 2) == 0

            @pl.when(jnp.logical_and(m > 1, even))
            def _():
                cscat_a_wait()

            @pl.when(jnp.logical_and(m > 1, jnp.logical_not(even)))
            def _():
                cscat_b_wait()

            @pl.when(even)
            def _():
                build_didx((2 * m + cid) * CHUNK, sidx_a)
                cscat_a_start()

            @pl.when(jnp.logical_not(even))
            def _():
                build_didx((2 * m + cid) * CHUNK, sidx_b)
                cscat_b_start()
            return 0

        for st, (n_real, n_pad) in enumerate(STAGES):
            e0 = sid * EPT + st * SEG
            pltpu.sync_copy(d_hbm.at[pl.ds(e0, n_real)],
                            dst_raw.at[pl.ds(0, n_real)])
            for t in range((n_pad - n_real) // L):
                dst_raw[pl.ds(n_real + t * L, L)] = dump16
            nmine = n_pad // (2 * CHUNK)
            lax.fori_loop(0, nmine, cnt_body, 0)
            cscat_a_wait()
            if nmine >= 2:
                cscat_b_wait()

        plsc.subcore_barrier()

        @pl.when(cid == 0)
        def _():
            pltpu.sync_copy(acc_sh.at[pl.ds(r0, WR_TILE)],
                            out_c0.at[pl.ds(r0, WR_TILE)])

        @pl.when(cid == 1)
        def _():
            pltpu.sync_copy(acc_sh.at[pl.ds(r0, WR_TILE)],
                            out_c1.at[pl.ds(r0, WR_TILE)])

        plsc.subcore_barrier()


def _sc_agg(x2, s_hbm, d_hbm):
    mesh = plsc.VectorSubcoreMesh(core_axis_name="c", subcore_axis_name="s")
    f32 = jnp.float32
    out_type = (
        jax.ShapeDtypeStruct((OUT_ROWS, DH), f32),
        jax.ShapeDtypeStruct((OUT_ROWS, DH), f32),
        jax.ShapeDtypeStruct((OUT_ROWS, DH), f32),
        jax.ShapeDtypeStruct((OUT_ROWS, DH), f32),
    )
    scratch = [
        pltpu.VMEM_SHARED((SH_ROWS, DH), f32),
        pltpu.VMEM((8, DH), f32),
        pltpu.VMEM((CHUNK, DH), f32),
        pltpu.VMEM((SEG,), jnp.int32),
        pltpu.VMEM((SEG,), jnp.int32),
        pltpu.VMEM((CHUNK, DH), f32),
        pltpu.VMEM((CHUNK, DH), f32),
        pltpu.VMEM((CHUNK,), jnp.int32),
        pltpu.VMEM((CHUNK,), jnp.int32),
        pltpu.VMEM((CHUNK,), jnp.int32),
        pltpu.VMEM((CHUNK,), jnp.int32),
        pltpu.SemaphoreType.DMA,
        pltpu.SemaphoreType.DMA,
        pltpu.SemaphoreType.DMA,
        pltpu.SemaphoreType.DMA,
    ]
    fn = pl.kernel(_sc_agg_kernel, out_type=out_type, mesh=mesh,
                   scratch_types=scratch)
    return fn(x2, s_hbm, d_hbm)


BM = 1000
NB = N // BM


def _tc_fused_kernel(alo_ref, ahi_ref, c0_ref, c1_ref, x_ref, wll_ref,
                     wlh_ref, wr_ref, bl_ref, gamma_ref, beta_ref,
                     out_ref, r_ref, stats_ref):
    p = pl.program_id(0)
    b = pl.program_id(1)

    @pl.when(p == 0)
    def _():
        cnt = (c0_ref[...][:, 0:1] - alo_ref[...][:, 0:1]
               + c1_ref[...][:, 0:1] - ahi_ref[...][:, 0:1])
        denom = jnp.maximum(cnt, 1.0)
        ml = alo_ref[...] / denom
        mh = ahi_ref[...] / denom
        h = (jnp.dot(ml, wll_ref[...], preferred_element_type=jnp.float32)
             + jnp.dot(mh, wlh_ref[...], preferred_element_type=jnp.float32)
             + jnp.dot(x_ref[...], wr_ref[...],
                       preferred_element_type=jnp.float32)
             + bl_ref[...])
        r = jnp.maximum(h, 0.0)
        r_ref[pl.ds(b * BM, BM), :] = r

        @pl.when(b == 0)
        def _():
            stats_ref[...] = jnp.zeros_like(stats_ref)

        stats_ref[0:1, :] += jnp.sum(r, axis=0, keepdims=True)
        stats_ref[1:2, :] += jnp.sum(r * r, axis=0, keepdims=True)

    @pl.when(p == 1)
    def _():
        m = stats_ref[0:1, :] / N
        var = stats_ref[1:2, :] / N - m * m
        inv = lax.rsqrt(var + EPS)
        scale = gamma_ref[...] * inv
        shift = beta_ref[...] - m * scale
        out_ref[...] = r_ref[pl.ds(b * BM, BM), :] * scale + shift


def _tc_post(agg_lo, agg_hi, cnt0, cnt1, x, wl, wr, bl, gamma, beta):
    f32 = jnp.float32
    wl_lo = wl[:DH]
    wl_hi = wl[DH:]
    out = pl.pallas_call(
        _tc_fused_kernel,
        grid=(2, NB),
        in_specs=[
            pl.BlockSpec((BM, DH), lambda p, b: (b, 0)),
            pl.BlockSpec((BM, DH), lambda p, b: (b, 0)),
            pl.BlockSpec((BM, DH), lambda p, b: (b, 0)),
            pl.BlockSpec((BM, DH), lambda p, b: (b, 0)),
            pl.BlockSpec((BM, D), lambda p, b: (b, 0)),
            pl.BlockSpec((DH, D), lambda p, b: (0, 0)),
            pl.BlockSpec((DH, D), lambda p, b: (0, 0)),
            pl.BlockSpec((D, D), lambda p, b: (0, 0)),
            pl.BlockSpec((1, D), lambda p, b: (0, 0)),
            pl.BlockSpec((1, D), lambda p, b: (0, 0)),
            pl.BlockSpec((1, D), lambda p, b: (0, 0)),
        ],
        out_specs=pl.BlockSpec((BM, D), lambda p, b: (b, 0)),
        out_shape=jax.ShapeDtypeStruct((N, D), f32),
        scratch_shapes=[
            pltpu.VMEM((N, D), f32),
            pltpu.VMEM((8, D), f32),
        ],
    )(agg_lo, agg_hi, cnt0, cnt1, x, wl_lo, wl_hi, wr, bl, gamma, beta)
    return out


def kernel(x_user, x_item, edge_index_u2i, edge_index_i2u,
           Wl_u2i, bl_u2i, Wr_u2i, Wl_i2u, bl_i2u, Wr_i2u,
           gamma_user, beta_user, gamma_item, beta_item):
    su2i, du2i = edge_index_u2i[0], edge_index_u2i[1]
    si2u, di2u = edge_index_i2u[0], edge_index_i2u[1]
    xu2 = x_user.reshape(2 * N, DH)
    xi2 = x_item.reshape(2 * N, DH)

    aggi_lo, aggi_hi, cnti0, cnti1 = _sc_agg(xu2, su2i, du2i)
    aggu_lo, aggu_hi, cntu0, cntu1 = _sc_agg(xi2, si2u, di2u)

    out_item = _tc_post(aggi_lo, aggi_hi, cnti0, cnti1, x_item,
                        Wl_u2i, Wr_u2i, bl_u2i.reshape(1, D),
                        gamma_item.reshape(1, D), beta_item.reshape(1, D))
    out_user = _tc_post(aggu_lo, aggu_hi, cntu0, cntu1, x_user,
                        Wl_i2u, Wr_i2u, bl_i2u.reshape(1, D),
                        gamma_user.reshape(1, D), beta_user.reshape(1, D))
    return (out_user, out_item)


# speedup vs baseline: 3.7890x; 1.0169x over previous
import jax
import jax.numpy as jnp
from jax import lax
from jax.experimental import pallas as pl
from jax.experimental.pallas import tpu as pltpu
from jax.experimental.pallas import tpu_sc as plsc

N = 10000
D = 256
E = 160000
EPS = 1e-5

NS = 16
L = 16
DH = 128
SH_ROWS = 10368
DUMP = 10240
OUT_ROWS = 10240
EPT = E // NS
CHUNK = 96
SEG = 1920
ZR_TILE = SH_ROWS // NS
WR_TILE = OUT_ROWS // NS
STAGES = ((1920, 1920), (1920, 1920), (1920, 1920), (1920, 1920),
          (1920, 1920), (400, 576))


def _sc_agg_kernel(x2, s_hbm, d_hbm,
                   agg_lo_o, agg_hi_o, cnt0_o, cnt1_o,
                   acc_sh, zrow, ones,
                   src_ea, dst_ea, src_eb, dst_eb, rows_a, rows_b,
                   gidx_a, gidx_b, sidx_a, sidx_b,
                   sem_a, sem_b, ssem_a, ssem_b, sem_e):
    cid = lax.axis_index("c")
    sid = lax.axis_index("s")
    z16 = jnp.zeros((L,), jnp.float32)
    zi16 = jnp.zeros((L,), jnp.int32)
    one16 = jnp.ones((L,), jnp.float32)
    dump16 = jnp.full((L,), DUMP, jnp.int32)

    def zrow_body(i, _):
        zrow[lax.rem(i, 8), pl.ds(lax.div(i, 8) * L, L)] = z16
        return 0
    lax.fori_loop(0, 8 * (DH // L), zrow_body, 0)

    def ones_body(i, _):
        ones[lax.rem(i, CHUNK), pl.ds(lax.div(i, CHUNK) * L, L)] = one16
        return 0
    lax.fori_loop(0, CHUNK * (DH // L), ones_body, 0)

    def zero_acc():
        def z_body(t, _):
            pltpu.sync_copy(zrow, acc_sh.at[pl.ds(sid * ZR_TILE + t * 8, 8)])
            return 0
        lax.fori_loop(0, ZR_TILE // 8, z_body, 0)

    for (out_lo, out_hi, out_c0, out_c1) in (
            (agg_lo_o, agg_hi_o, cnt0_o, cnt1_o),):
        zero_acc()
        plsc.subcore_barrier()

        def build_idx(src_raw, dst_raw, base, gidx, sidx):
            for t in range(CHUNK // L):
                s = src_raw[pl.ds(base + t * L, L)]
                d = dst_raw[pl.ds(base + t * L, L)]
                gidx[pl.ds(t * L, L)] = 2 * s + cid
                sidx[pl.ds(t * L, L)] = d

        def gather_a():
            return pltpu.make_async_copy(x2.at[gidx_a], rows_a, sem_a)

        def gather_b():
            return pltpu.make_async_copy(x2.at[gidx_b], rows_b, sem_b)

        def scatter_a_start():
            pltpu.async_copy(rows_a, acc_sh.at[sidx_a], ssem_a, add=True)

        def scatter_a_wait():
            pltpu.make_async_copy(rows_a, acc_sh.at[sidx_a], ssem_a).wait()

        def scatter_b_start():
            pltpu.async_copy(rows_b, acc_sh.at[sidx_b], ssem_b, add=True)

        def scatter_b_wait():
            pltpu.make_async_copy(rows_b, acc_sh.at[sidx_b], ssem_b).wait()

        ebufs = ((src_ea, dst_ea), (src_eb, dst_eb))
        pltpu.sync_copy(s_hbm.at[pl.ds(sid * EPT, STAGES[0][0])],
                        src_ea.at[pl.ds(0, STAGES[0][0])])
        pltpu.sync_copy(d_hbm.at[pl.ds(sid * EPT, STAGES[0][0])],
                        dst_ea.at[pl.ds(0, STAGES[0][0])])
        for st, (n_real, n_pad) in enumerate(STAGES):
            src_raw, dst_raw = ebufs[st % 2]
            if st + 1 < len(STAGES):
                nsrc, ndst = ebufs[(st + 1) % 2]
                nn = STAGES[st + 1][0]
                e1 = sid * EPT + (st + 1) * SEG
                pltpu.async_copy(s_hbm.at[pl.ds(e1, nn)],
                                 nsrc.at[pl.ds(0, nn)], sem_e)
                pltpu.async_copy(d_hbm.at[pl.ds(e1, nn)],
                                 ndst.at[pl.ds(0, nn)], sem_e)
            for t in range((n_pad - n_real) // L):
                src_raw[pl.ds(n_real + t * L, L)] = zi16
                dst_raw[pl.ds(n_real + t * L, L)] = dump16

            npairs = n_pad // (2 * CHUNK)
            build_idx(src_raw, dst_raw, 0, gidx_a, sidx_a)
            gather_a().start()

            def pair_body(p, _):
                @pl.when(p > 0)
                def _():
                    scatter_b_wait()
                build_idx(src_raw, dst_raw, (2 * p + 1) * CHUNK,
                          gidx_b, sidx_b)
                gather_b().start()
                gather_a().wait()
                scatter_a_start()

                @pl.when(p < npairs - 1)
                def _():
                    scatter_a_wait()
                    build_idx(src_raw, dst_raw, (2 * p + 2) * CHUNK,
                              gidx_a, sidx_a)
                    gather_a().start()

                gather_b().wait()
                scatter_b_start()
                return 0
            lax.fori_loop(0, npairs, pair_body, 0)
            scatter_a_wait()
            scatter_b_wait()
            if st + 1 < len(STAGES):
                nsrc, ndst = ebufs[(st + 1) % 2]
                nn = STAGES[st + 1][0]
                e1 = sid * EPT + (st + 1) * SEG
                pltpu.make_async_copy(s_hbm.at[pl.ds(e1, nn)],
                                      nsrc.at[pl.ds(0, nn)], sem_e).wait()
                pltpu.make_async_copy(d_hbm.at[pl.ds(e1, nn)],
                                      ndst.at[pl.ds(0, nn)], sem_e).wait()

        plsc.subcore_barrier()

        r0 = sid * WR_TILE

        @pl.when(cid == 0)
        def _():
            pltpu.sync_copy(acc_sh.at[pl.ds(r0, WR_TILE)],
                            out_lo.at[pl.ds(r0, WR_TILE)])

        @pl.when(cid == 1)
        def _():
            pltpu.sync_copy(acc_sh.at[pl.ds(r0, WR_TILE)],
                            out_hi.at[pl.ds(r0, WR_TILE)])

        plsc.subcore_barrier()


        def cscat_a_start():
            pltpu.async_copy(ones, acc_sh.at[sidx_a], ssem_a, add=True)

        def cscat_a_wait():
            pltpu.make_async_copy(ones, acc_sh.at[sidx_a], ssem_a).wait()

        def cscat_b_start():
            pltpu.async_copy(ones, acc_sh.at[sidx_b], ssem_b, add=True)

        def cscat_b_wait():
            pltpu.make_async_copy(ones, acc_sh.at[sidx_b], ssem_b).wait()

        def build_didx(base, sidx):
            for t in range(CHUNK // L):
                sidx[pl.ds(t * L, L)] = dst_ea[pl.ds(base + t * L, L)]

        def cnt_body(m, _):
            even = lax.rem(m, 2) == 0

            @pl.when(jnp.logical_and(m > 1, even))
            def _():
                cscat_a_wait()

            @pl.when(jnp.logical_and(m > 1, jnp.logical_not(even)))
            def _():
                cscat_b_wait()

            @pl.when(even)
            def _():
                build_didx((2 * m + cid) * CHUNK, sidx_a)
                cscat_a_start()

            @pl.when(jnp.logical_not(even))
            def _():
                build_didx((2 * m + cid) * CHUNK, sidx_b)
                cscat_b_start()
            return 0

        for st, (n_real, n_pad) in enumerate(STAGES):
            e0 = sid * EPT + st * SEG
            pltpu.sync_copy(d_hbm.at[pl.ds(e0, n_real)],
                            dst_ea.at[pl.ds(0, n_real)])
            for t in range((n_pad - n_real) // L):
                dst_ea[pl.ds(n_real + t * L, L)] = dump16
            nmine = n_pad // (2 * CHUNK)
            lax.fori_loop(0, nmine, cnt_body, 0)
            cscat_a_wait()
            if nmine >= 2:
                cscat_b_wait()

        plsc.subcore_barrier()

        @pl.when(cid == 0)
        def _():
            pltpu.sync_copy(acc_sh.at[pl.ds(r0, WR_TILE)],
                            out_c0.at[pl.ds(r0, WR_TILE)])

        @pl.when(cid == 1)
        def _():
            pltpu.sync_copy(acc_sh.at[pl.ds(r0, WR_TILE)],
                            out_c1.at[pl.ds(r0, WR_TILE)])

        plsc.subcore_barrier()


def _sc_agg(x2, s_hbm, d_hbm):
    mesh = plsc.VectorSubcoreMesh(core_axis_name="c", subcore_axis_name="s")
    f32 = jnp.float32
    out_type = (
        jax.ShapeDtypeStruct((OUT_ROWS, DH), f32),
        jax.ShapeDtypeStruct((OUT_ROWS, DH), f32),
        jax.ShapeDtypeStruct((OUT_ROWS, DH), f32),
        jax.ShapeDtypeStruct((OUT_ROWS, DH), f32),
    )
    scratch = [
        pltpu.VMEM_SHARED((SH_ROWS, DH), f32),
        pltpu.VMEM((8, DH), f32),
        pltpu.VMEM((CHUNK, DH), f32),
        pltpu.VMEM((SEG,), jnp.int32),
        pltpu.VMEM((SEG,), jnp.int32),
        pltpu.VMEM((SEG,), jnp.int32),
        pltpu.VMEM((SEG,), jnp.int32),
        pltpu.VMEM((CHUNK, DH), f32),
        pltpu.VMEM((CHUNK, DH), f32),
        pltpu.VMEM((CHUNK,), jnp.int32),
        pltpu.VMEM((CHUNK,), jnp.int32),
        pltpu.VMEM((CHUNK,), jnp.int32),
        pltpu.VMEM((CHUNK,), jnp.int32),
        pltpu.SemaphoreType.DMA,
        pltpu.SemaphoreType.DMA,
        pltpu.SemaphoreType.DMA,
        pltpu.SemaphoreType.DMA,
        pltpu.SemaphoreType.DMA,
    ]
    fn = pl.kernel(_sc_agg_kernel, out_type=out_type, mesh=mesh,
                   scratch_types=scratch)
    return fn(x2, s_hbm, d_hbm)


BM = 1000
NB = N // BM


def _tc_fused_kernel(alo_ref, ahi_ref, c0_ref, c1_ref, x_ref, wll_ref,
                     wlh_ref, wr_ref, bl_ref, gamma_ref, beta_ref,
                     out_ref, r_ref, stats_ref):
    p = pl.program_id(0)
    b = pl.program_id(1)

    @pl.when(p == 0)
    def _():
        cnt = (c0_ref[...][:, 0:1] - alo_ref[...][:, 0:1]
               + c1_ref[...][:, 0:1] - ahi_ref[...][:, 0:1])
        denom = jnp.maximum(cnt, 1.0)
        ml = alo_ref[...] / denom
        mh = ahi_ref[...] / denom
        bf = jnp.bfloat16
        h = (jnp.dot(ml.astype(bf), wll_ref[...].astype(bf),
                     preferred_element_type=jnp.float32)
             + jnp.dot(mh.astype(bf), wlh_ref[...].astype(bf),
                       preferred_element_type=jnp.float32)
             + jnp.dot(x_ref[...].astype(bf), wr_ref[...].astype(bf),
                       preferred_element_type=jnp.float32)
             + bl_ref[...])
        r = jnp.maximum(h, 0.0)
        r_ref[pl.ds(b * BM, BM), :] = r

        @pl.when(b == 0)
        def _():
            stats_ref[...] = jnp.zeros_like(stats_ref)

        stats_ref[0:1, :] += jnp.sum(r, axis=0, keepdims=True)
        stats_ref[1:2, :] += jnp.sum(r * r, axis=0, keepdims=True)

    @pl.when(p == 1)
    def _():
        m = stats_ref[0:1, :] / N
        var = stats_ref[1:2, :] / N - m * m
        inv = lax.rsqrt(var + EPS)
        scale = gamma_ref[...] * inv
        shift = beta_ref[...] - m * scale
        out_ref[...] = r_ref[pl.ds(b * BM, BM), :] * scale + shift


def _tc_post(agg_lo, agg_hi, cnt0, cnt1, x, wl, wr, bl, gamma, beta):
    f32 = jnp.float32
    wl_lo = wl[:DH]
    wl_hi = wl[DH:]
    out = pl.pallas_call(
        _tc_fused_kernel,
        grid=(2, NB),
        in_specs=[
            pl.BlockSpec((BM, DH), lambda p, b: (b, 0)),
            pl.BlockSpec((BM, DH), lambda p, b: (b, 0)),
            pl.BlockSpec((BM, DH), lambda p, b: (b, 0)),
            pl.BlockSpec((BM, DH), lambda p, b: (b, 0)),
            pl.BlockSpec((BM, D), lambda p, b: (b, 0)),
            pl.BlockSpec((DH, D), lambda p, b: (0, 0)),
            pl.BlockSpec((DH, D), lambda p, b: (0, 0)),
            pl.BlockSpec((D, D), lambda p, b: (0, 0)),
            pl.BlockSpec((1, D), lambda p, b: (0, 0)),
            pl.BlockSpec((1, D), lambda p, b: (0, 0)),
            pl.BlockSpec((1, D), lambda p, b: (0, 0)),
        ],
        out_specs=pl.BlockSpec((BM, D), lambda p, b: (b, 0)),
        out_shape=jax.ShapeDtypeStruct((N, D), f32),
        scratch_shapes=[
            pltpu.VMEM((N, D), f32),
            pltpu.VMEM((8, D), f32),
        ],
    )(agg_lo, agg_hi, cnt0, cnt1, x, wl_lo, wl_hi, wr, bl, gamma, beta)
    return out


def kernel(x_user, x_item, edge_index_u2i, edge_index_i2u,
           Wl_u2i, bl_u2i, Wr_u2i, Wl_i2u, bl_i2u, Wr_i2u,
           gamma_user, beta_user, gamma_item, beta_item):
    su2i, du2i = edge_index_u2i[0], edge_index_u2i[1]
    si2u, di2u = edge_index_i2u[0], edge_index_i2u[1]
    xu2 = x_user.reshape(2 * N, DH)
    xi2 = x_item.reshape(2 * N, DH)

    aggi_lo, aggi_hi, cnti0, cnti1 = _sc_agg(xu2, su2i, du2i)
    aggu_lo, aggu_hi, cntu0, cntu1 = _sc_agg(xi2, si2u, di2u)

    out_item = _tc_post(aggi_lo, aggi_hi, cnti0, cnti1, x_item,
                        Wl_u2i, Wr_u2i, bl_u2i.reshape(1, D),
                        gamma_item.reshape(1, D), beta_item.reshape(1, D))
    out_user = _tc_post(aggu_lo, aggu_hi, cntu0, cntu1, x_user,
                        Wl_i2u, Wr_i2u, bl_i2u.reshape(1, D),
                        gamma_user.reshape(1, D), beta_user.reshape(1, D))
    return (out_user, out_item)
